# baseline (speedup 1.0000x reference)
import functools

import jax
import jax.numpy as jnp
from jax import lax
from jax.experimental import pallas as pl
from jax.experimental.pallas import tpu as pltpu
from jax.experimental.pallas import tpu_sc as plsc

_N = 10000
_K = 40
_NP = 10240
_E = _N * _K
_EPAD = 409600
_PB = 200
_EB = _PB * _K
_RB = 512
_SC_CH = 1280


def _leaky(x):
    return jnp.where(x >= 0, x, 0.2 * x)


_M = 12


def _knn_body(x2c_ref, q_ref, qn_ref, xt_ref, out_ref, d2_ref):
    i = pl.program_id(0)
    rb = q_ref.shape[0]
    npw = x2c_ref.shape[1]
    nch = npw // 128
    mm = jnp.dot(q_ref[...], xt_ref[...])
    d2 = (qn_ref[...] + x2c_ref[...]) - 2.0 * mm
    col = lax.broadcasted_iota(jnp.int32, d2.shape, 1)
    row = i * rb + lax.broadcasted_iota(jnp.int32, d2.shape, 0)
    d2_ref[...] = jnp.where(col == row, jnp.inf, d2)

    lane = lax.broadcasted_iota(jnp.int32, (8, 128), 1)
    big = jnp.int32(2 ** 30)

    def rg_body(rg, _):
        r0 = rg * 8
        vacc = [jnp.full((8, 128), jnp.inf, jnp.float32) for _ in range(_M)]
        iacc = [jnp.full((8, 128), 0, jnp.int32) for _ in range(_M)]
        for c in range(nch):
            v = d2_ref[pl.ds(r0, 8), pl.ds(c * 128, 128)]
            ix = lane + (c * 128)
            for j in range(_M):
                lt = v < vacc[j]
                vn = jnp.where(lt, v, vacc[j])
                v = jnp.where(lt, vacc[j], v)
                jn = jnp.where(lt, ix, iacc[j])
                ix = jnp.where(lt, iacc[j], ix)
                vacc[j], iacc[j] = vn, jn

        heads, hidx = vacc[0], iacc[0]
        pos = jnp.zeros((8, 128), jnp.int32)
        out = jnp.zeros((8, 128), jnp.int32)
        for t in range(_K):
            m = jnp.min(heads, axis=1)
            hit = heads == m[:, None]
            a = jnp.min(jnp.where(hit, hidx, big), axis=1)
            sel = hit & (hidx == a[:, None])
            out = jnp.where(lane == t, a[:, None], out)
            pos = pos + sel.astype(jnp.int32)
            nh = jnp.full((8, 128), jnp.inf, jnp.float32)
            ni = jnp.zeros((8, 128), jnp.int32)
            for j in range(1, _M):
                eq = pos == j
                nh = jnp.where(eq, vacc[j], nh)
                ni = jnp.where(eq, iacc[j], ni)
            heads = jnp.where(sel, nh, heads)
            hidx = jnp.where(sel, ni, hidx)
        out_ref[pl.ds(r0, 8), :] = out
        return 0

    lax.fori_loop(0, rb // 8, rg_body, 0)


def _knn(xp, x2):
    npad, dp = xp.shape
    return pl.pallas_call(
        _knn_body,
        grid=(npad // _RB,),
        in_specs=[
            pl.BlockSpec((1, npad), lambda i: (0, 0)),
            pl.BlockSpec((_RB, dp), lambda i: (i, 0)),
            pl.BlockSpec((_RB, 1), lambda i: (i, 0)),
            pl.BlockSpec((dp, npad), lambda i: (0, 0)),
        ],
        out_specs=pl.BlockSpec((_RB, 128), lambda i: (i, 0)),
        out_shape=jax.ShapeDtypeStruct((npad, 128), jnp.int32),
        scratch_shapes=[pltpu.VMEM((_RB, npad), jnp.float32)],
    )(x2[None, :], xp, x2[:, None], xp.T)


@functools.lru_cache(maxsize=None)
def _make_gather(dpsc):
    per_w = _EPAD // 32
    n_ch = per_w // _SC_CH
    mesh = plsc.VectorSubcoreMesh(core_axis_name="c", subcore_axis_name="s")

    @functools.partial(
        pl.kernel, mesh=mesh,
        compiler_params=pltpu.CompilerParams(use_tc_tiling_on_sc=False),
        out_type=jax.ShapeDtypeStruct((_EPAD, dpsc), jnp.float32),
        scratch_types=[
            pltpu.VMEM((_SC_CH,), jnp.int32),
            pltpu.VMEM((_SC_CH, dpsc), jnp.float32),
            pltpu.SemaphoreType.DMA,
        ],
    )
    def gather(table_hbm, idx_hbm, out_hbm, idx_v, rows_v, sem):
        wid = lax.axis_index("s") * 2 + lax.axis_index("c")
        base = wid * per_w

        def body(c, carry):
            off = base + c * _SC_CH
            pltpu.sync_copy(idx_hbm.at[pl.ds(off, _SC_CH)], idx_v)
            pltpu.async_copy(table_hbm.at[idx_v], rows_v, sem).wait()
            pltpu.sync_copy(rows_v, out_hbm.at[pl.ds(off, _SC_CH)])
            return carry

        lax.fori_loop(0, n_ch, body, 0)

    return gather


def _gather_rows(table, idxf):
    return _make_gather(table.shape[1])(table, idxf)


def _edges(g_ref, xb_ref, w1_ref, b1_ref):
    xb = xb_ref[...]
    pb, dp = xb.shape
    xi = jnp.broadcast_to(xb[:, None, :], (pb, _K, dp)).reshape(pb * _K, dp)
    e = jnp.concatenate([xi, g_ref[...] - xi], axis=-1)
    return jnp.dot(e, w1_ref[...]) + b1_ref[...]


def _p1_body(g_ref, xb_ref, w1_ref, b1_ref, s_ref, ss_ref):
    h1 = _edges(g_ref, xb_ref, w1_ref, b1_ref)

    @pl.when(pl.program_id(0) == 0)
    def _():
        s_ref[...] = jnp.zeros_like(s_ref)
        ss_ref[...] = jnp.zeros_like(ss_ref)

    s_ref[...] += jnp.sum(h1, axis=0, keepdims=True)
    ss_ref[...] += jnp.sum(h1 * h1, axis=0, keepdims=True)


def _bn(h, bn_ref):
    m, v = bn_ref[0:1, :], bn_ref[1:2, :]
    gm, bt = bn_ref[2:3, :], bn_ref[3:4, :]
    return _leaky(gm * (h - m) / jnp.sqrt(v + 1e-5) + bt)


def _p2_body(g_ref, xb_ref, w1_ref, b1_ref, bn1_ref, w2_ref, b2_ref,
             h2_ref, s_ref, ss_ref):
    u = _bn(_edges(g_ref, xb_ref, w1_ref, b1_ref), bn1_ref)
    h2 = jnp.dot(u, w2_ref[...]) + b2_ref[...]
    h2_ref[...] = h2

    @pl.when(pl.program_id(0) == 0)
    def _():
        s_ref[...] = jnp.zeros_like(s_ref)
        ss_ref[...] = jnp.zeros_like(ss_ref)

    s_ref[...] += jnp.sum(h2, axis=0, keepdims=True)
    ss_ref[...] += jnp.sum(h2 * h2, axis=0, keepdims=True)


def _p3_pool_body(h2_ref, bn2_ref, o_ref):
    u = _bn(h2_ref[...], bn2_ref)
    c2 = u.shape[-1]
    o_ref[...] = jnp.max(u.reshape(_PB, _K, c2), axis=1)


def _p3_epi_body(h2_ref, bn2_ref, xb_ref, ltw_ref, ltb_ref, l0w_ref, l0b_ref,
                 l1w_ref, l1b_ref, o_ref):
    u = _bn(h2_ref[...], bn2_ref)
    c2 = u.shape[-1]
    o = jnp.max(u.reshape(_PB, _K, c2), axis=1)
    h = jnp.maximum(jnp.dot(o, l0w_ref[...]) + l0b_ref[...], 0.0)
    xi = jnp.dot(h, l1w_ref[...]) + l1b_ref[...]
    o_ref[...] = (jnp.dot(xb_ref[...], ltw_ref[...]) + ltb_ref[...]) + xi


def _full2(a):
    return pl.BlockSpec(a.shape, lambda i: (0, 0))


def _conv_block(x, layers, epi):
    n, d = x.shape
    dpsc = {3: 16, 19: 32, 64: 64}[d]
    xp = jnp.pad(x, ((0, _NP - n), (0, dpsc - d)))
    x2 = jnp.pad(jnp.sum(x * x, axis=-1), (0, _NP - n),
                 constant_values=jnp.inf)
    idxk = _knn(xp, x2)
    idxf = jnp.pad(idxk[:n, :_K].reshape(-1), (0, _EPAD - _E))
    g = _gather_rows(xp, idxf)[:_E]

    w1 = layers[0]['W']
    c1 = w1.shape[1]
    w1p = jnp.zeros((2 * dpsc, c1), jnp.float32)
    w1p = w1p.at[:d].set(w1[:d]).at[dpsc:dpsc + d].set(w1[d:])
    b1 = layers[0]['b'][None]
    xb = xp[:n]
    grid = (_E // _EB,)
    gspec = pl.BlockSpec((_EB, dpsc), lambda i: (i, 0))
    xspec = pl.BlockSpec((_PB, dpsc), lambda i: (i, 0))
    stat = pl.BlockSpec((1, c1), lambda i: (0, 0))

    s1, ss1 = pl.pallas_call(
        _p1_body, grid=grid,
        in_specs=[gspec, xspec, _full2(w1p), _full2(b1)],
        out_specs=[stat, stat],
        out_shape=[jax.ShapeDtypeStruct((1, c1), jnp.float32)] * 2,
    )(g, xb, w1p, b1)
    m1 = s1 / _E
    v1 = ss1 / _E - m1 * m1
    bn1 = jnp.concatenate(
        [m1, v1, layers[0]['gamma'][None], layers[0]['beta'][None]], 0)

    w2 = layers[1]['W']
    c2 = w2.shape[1]
    b2 = layers[1]['b'][None]
    stat2 = pl.BlockSpec((1, c2), lambda i: (0, 0))
    h2, s2, ss2 = pl.pallas_call(
        _p2_body, grid=grid,
        in_specs=[gspec, xspec, _full2(w1p), _full2(b1), _full2(bn1),
                  _full2(w2), _full2(b2)],
        out_specs=[pl.BlockSpec((_EB, c2), lambda i: (i, 0)), stat2, stat2],
        out_shape=[jax.ShapeDtypeStruct((_E, c2), jnp.float32),
                   jax.ShapeDtypeStruct((1, c2), jnp.float32),
                   jax.ShapeDtypeStruct((1, c2), jnp.float32)],
    )(g, xb, w1p, b1, bn1, w2, b2)
    m2 = s2 / _E
    v2 = ss2 / _E - m2 * m2
    bn2 = jnp.concatenate(
        [m2, v2, layers[1]['gamma'][None], layers[1]['beta'][None]], 0)

    h2spec = pl.BlockSpec((_EB, c2), lambda i: (i, 0))
    ospec = pl.BlockSpec((_PB, c2), lambda i: (i, 0))
    if epi is None:
        return pl.pallas_call(
            _p3_pool_body, grid=grid,
            in_specs=[h2spec, _full2(bn2)],
            out_specs=ospec,
            out_shape=jax.ShapeDtypeStruct((n, c2), jnp.float32),
        )(h2, bn2)

    lt, l0, l1 = epi
    ltwp = jnp.zeros((dpsc, c2), jnp.float32).at[:d].set(lt['W'])
    return pl.pallas_call(
        _p3_epi_body, grid=grid,
        in_specs=[h2spec, _full2(bn2), xspec, _full2(ltwp),
                  _full2(lt['b'][None]), _full2(l0['W']),
                  _full2(l0['b'][None]), _full2(l1['W']),
                  _full2(l1['b'][None])],
        out_specs=ospec,
        out_shape=jax.ShapeDtypeStruct((n, c2), jnp.float32),
    )(h2, bn2, xb, ltwp, lt['b'][None], l0['W'], l0['b'][None],
      l1['W'], l1['b'][None])


def _t2_body(x_ref, w_ref, b_ref, s_ref, ss_ref, mx_ref, mn_ref):
    y = jnp.dot(x_ref[...], w_ref[...]) + b_ref[...]

    @pl.when(pl.program_id(0) == 0)
    def _():
        s_ref[...] = jnp.zeros_like(s_ref)
        ss_ref[...] = jnp.zeros_like(ss_ref)
        mx_ref[...] = jnp.full_like(mx_ref, -jnp.inf)
        mn_ref[...] = jnp.full_like(mn_ref, jnp.inf)

    s_ref[...] += jnp.sum(y, axis=0, keepdims=True)
    ss_ref[...] += jnp.sum(y * y, axis=0, keepdims=True)
    mx_ref[...] = jnp.maximum(mx_ref[...], jnp.max(y, axis=0, keepdims=True))
    mn_ref[...] = jnp.minimum(mn_ref[...], jnp.min(y, axis=0, keepdims=True))


def _t2_pool(x1, p):
    n, din = x1.shape
    dout = p['W'].shape[1]
    rb = 1000
    stat = pl.BlockSpec((1, dout), lambda i: (0, 0))
    s, ss, mx, mn = pl.pallas_call(
        _t2_body, grid=(n // rb,),
        in_specs=[pl.BlockSpec((rb, din), lambda i: (i, 0)),
                  _full2(p['W']), _full2(p['b'][None])],
        out_specs=[stat] * 4,
        out_shape=[jax.ShapeDtypeStruct((1, dout), jnp.float32)] * 4,
    )(x1, p['W'], p['b'][None])
    m = s / n
    v = ss / n - m * m
    g = p['gamma'][None]
    pooled = jnp.where(g >= 0, mx, mn)
    return _leaky(g * (pooled - m) / jnp.sqrt(v + 1e-5) + p['beta'][None])


def _head_body(p_ref, w1_ref, b1_ref, w2_ref, b2_ref, w4_ref, b4_ref, o_ref):
    h = _leaky(jnp.dot(p_ref[...], w1_ref[...]) + b1_ref[...])
    h = _leaky(jnp.dot(h, w2_ref[...]) + b2_ref[...])
    o_ref[...] = jnp.dot(h, w4_ref[...]) + b4_ref[...]


def _head(pooled, t3, t4):
    args = (pooled, t3[0]['W'], t3[0]['b'][None], t3[1]['W'], t3[1]['b'][None],
            t4['W'], t4['b'][None])
    return pl.pallas_call(
        _head_body,
        in_specs=[pl.BlockSpec(a.shape, lambda: (0, 0)) for a in args],
        out_specs=pl.BlockSpec((1, 9), lambda: (0, 0)),
        out_shape=jax.ShapeDtypeStruct((1, 9), jnp.float32),
    )(*args)


def kernel(positions, features, batch_indices, params):
    del batch_indices
    x1 = _conv_block(positions, params['t1'], None)
    pooled = _t2_pool(x1, params['t2'][0])
    x9 = _head(pooled, params['t3'], params['t4'])
    t = x9.reshape(3, 3)
    x0 = (positions[:, 0:1] * t[0][None, :]
          + positions[:, 1:2] * t[1][None, :]
          + positions[:, 2:3] * t[2][None, :])
    x = jnp.concatenate([x0, features], axis=-1)
    for i in range(2):
        x = _conv_block(x, params['convs'][i],
                        (params['lin_transform'][i],) +
                        tuple(params['lin_layers'][i]))
    return x


# speedup vs baseline: 5.7867x; 5.7867x over previous
import functools

import jax
import jax.numpy as jnp
from jax import lax
from jax.experimental import pallas as pl
from jax.experimental.pallas import tpu as pltpu
from jax.experimental.pallas import tpu_sc as plsc

_N = 10000
_K = 40
_NP = 10240
_E = _N * _K
_EPAD = 409600
_PB = 200
_EB = _PB * _K
_RB = 512
_SC_CH = 1280


def _leaky(x):
    return jnp.where(x >= 0, x, 0.2 * x)


_M = 10


def _knn_body(x2c_ref, q_ref, qn_ref, xt_ref, out_ref,
              d2_ref, cv_ref, ci_ref):
    i = pl.program_id(0)
    rb = q_ref.shape[0]
    npw = x2c_ref.shape[1]
    nch = npw // 128
    mm = jnp.dot(q_ref[...], xt_ref[...])
    d2 = (qn_ref[...] + x2c_ref[...]) - 2.0 * mm
    col = lax.broadcasted_iota(jnp.int32, d2.shape, 1)
    row = i * rb + lax.broadcasted_iota(jnp.int32, d2.shape, 0)
    d2_ref[...] = jnp.where(col == row, jnp.inf, d2)

    lane = lax.broadcasted_iota(jnp.int32, (8, 128), 1)

    def rg_body(rg, _):
        r0 = rg * 8
        vacc = [jnp.full((8, 128), jnp.inf, jnp.float32) for _ in range(_M)]
        iacc = [jnp.full((8, 128), 0, jnp.int32) for _ in range(_M)]
        for c in range(nch):
            v = d2_ref[pl.ds(r0, 8), pl.ds(c * 128, 128)]
            ix = lane + (c * 128)
            for j in range(_M):
                lt = v < vacc[j]
                vn = jnp.where(lt, v, vacc[j])
                v = jnp.where(lt, vacc[j], v)
                jn = jnp.where(lt, ix, iacc[j])
                ix = jnp.where(lt, iacc[j], ix)
                vacc[j], iacc[j] = vn, jn
        for j in range(_M):
            cv_ref[pl.ds(r0, 8), pl.ds(j * 128, 128)] = vacc[j]
            ci_ref[pl.ds(r0, 8), pl.ds(j * 128, 128)] = iacc[j]
        return 0

    lax.fori_loop(0, rb // 8, rg_body, 0)

    def step(t, _):
        cv = cv_ref[...]
        m = jnp.min(cv, axis=1)
        ci = ci_ref[...]
        hit = cv == m[:, None]
        a = jnp.min(jnp.where(hit, ci, jnp.int32(2**30)), axis=1)
        a = a.astype(jnp.int32)
        out_ref[pl.ds(t, 1), :] = a[None, :]
        cv_ref[...] = jnp.where(hit & (ci == a[:, None]), jnp.inf, cv)
        return 0

    lax.fori_loop(0, _K, step, 0, unroll=4)


def _knn(xp, x2):
    npad, dp = xp.shape
    return pl.pallas_call(
        _knn_body,
        grid=(npad // _RB,),
        in_specs=[
            pl.BlockSpec((1, npad), lambda i: (0, 0)),
            pl.BlockSpec((_RB, dp), lambda i: (i, 0)),
            pl.BlockSpec((_RB, 1), lambda i: (i, 0)),
            pl.BlockSpec((dp, npad), lambda i: (0, 0)),
        ],
        out_specs=pl.BlockSpec((_K, _RB), lambda i: (0, i)),
        out_shape=jax.ShapeDtypeStruct((_K, npad), jnp.int32),
        scratch_shapes=[pltpu.VMEM((_RB, npad), jnp.float32),
                        pltpu.VMEM((_RB, _M * 128), jnp.float32),
                        pltpu.VMEM((_RB, _M * 128), jnp.int32)],
    )(x2[None, :], xp, x2[:, None], xp.T)


@functools.lru_cache(maxsize=None)
def _make_gather(dpsc):
    per_w = _EPAD // 32
    n_ch = per_w // _SC_CH
    mesh = plsc.VectorSubcoreMesh(core_axis_name="c", subcore_axis_name="s")

    @functools.partial(
        pl.kernel, mesh=mesh,
        compiler_params=pltpu.CompilerParams(use_tc_tiling_on_sc=False),
        out_type=jax.ShapeDtypeStruct((_EPAD, dpsc), jnp.float32),
        scratch_types=[
            pltpu.VMEM((_SC_CH,), jnp.int32),
            pltpu.VMEM((_SC_CH, dpsc), jnp.float32),
            pltpu.SemaphoreType.DMA,
        ],
    )
    def gather(table_hbm, idx_hbm, out_hbm, idx_v, rows_v, sem):
        wid = lax.axis_index("s") * 2 + lax.axis_index("c")
        base = wid * per_w

        def body(c, carry):
            off = base + c * _SC_CH
            pltpu.sync_copy(idx_hbm.at[pl.ds(off, _SC_CH)], idx_v)
            pltpu.async_copy(table_hbm.at[idx_v], rows_v, sem).wait()
            pltpu.sync_copy(rows_v, out_hbm.at[pl.ds(off, _SC_CH)])
            return carry

        lax.fori_loop(0, n_ch, body, 0)

    return gather


def _gather_rows(table, idxf):
    return _make_gather(table.shape[1])(table, idxf)


def _edges(g_ref, xb_ref, w1_ref, b1_ref):
    xb = xb_ref[...]
    pb, dp = xb.shape
    xi = jnp.broadcast_to(xb[:, None, :], (pb, _K, dp)).reshape(pb * _K, dp)
    e = jnp.concatenate([xi, g_ref[...] - xi], axis=-1)
    return jnp.dot(e, w1_ref[...]) + b1_ref[...]


def _p1_body(g_ref, xb_ref, w1_ref, b1_ref, s_ref, ss_ref):
    h1 = _edges(g_ref, xb_ref, w1_ref, b1_ref)

    @pl.when(pl.program_id(0) == 0)
    def _():
        s_ref[...] = jnp.zeros_like(s_ref)
        ss_ref[...] = jnp.zeros_like(ss_ref)

    s_ref[...] += jnp.sum(h1, axis=0, keepdims=True)
    ss_ref[...] += jnp.sum(h1 * h1, axis=0, keepdims=True)


def _bn(h, bn_ref):
    m, v = bn_ref[0:1, :], bn_ref[1:2, :]
    gm, bt = bn_ref[2:3, :], bn_ref[3:4, :]
    return _leaky(gm * (h - m) / jnp.sqrt(v + 1e-5) + bt)


def _p2_body(g_ref, xb_ref, w1_ref, b1_ref, bn1_ref, w2_ref, b2_ref,
             h2_ref, s_ref, ss_ref):
    u = _bn(_edges(g_ref, xb_ref, w1_ref, b1_ref), bn1_ref)
    h2 = jnp.dot(u, w2_ref[...]) + b2_ref[...]
    h2_ref[...] = h2

    @pl.when(pl.program_id(0) == 0)
    def _():
        s_ref[...] = jnp.zeros_like(s_ref)
        ss_ref[...] = jnp.zeros_like(ss_ref)

    s_ref[...] += jnp.sum(h2, axis=0, keepdims=True)
    ss_ref[...] += jnp.sum(h2 * h2, axis=0, keepdims=True)


def _p3_pool_body(h2_ref, bn2_ref, o_ref):
    u = _bn(h2_ref[...], bn2_ref)
    c2 = u.shape[-1]
    o_ref[...] = jnp.max(u.reshape(_PB, _K, c2), axis=1)


def _p3_epi_body(h2_ref, bn2_ref, xb_ref, ltw_ref, ltb_ref, l0w_ref, l0b_ref,
                 l1w_ref, l1b_ref, o_ref):
    u = _bn(h2_ref[...], bn2_ref)
    c2 = u.shape[-1]
    o = jnp.max(u.reshape(_PB, _K, c2), axis=1)
    h = jnp.maximum(jnp.dot(o, l0w_ref[...]) + l0b_ref[...], 0.0)
    xi = jnp.dot(h, l1w_ref[...]) + l1b_ref[...]
    o_ref[...] = (jnp.dot(xb_ref[...], ltw_ref[...]) + ltb_ref[...]) + xi


def _full2(a):
    return pl.BlockSpec(a.shape, lambda i: (0, 0))


def _conv_block(x, layers, epi):
    n, d = x.shape
    dpsc = {3: 16, 19: 32, 64: 64}[d]
    xp = jnp.pad(x, ((0, _NP - n), (0, dpsc - d)))
    x2 = jnp.pad(jnp.sum(x * x, axis=-1), (0, _NP - n),
                 constant_values=jnp.inf)
    idxk = _knn(xp, x2)
    idxf = jnp.pad(idxk.T[:n].reshape(-1), (0, _EPAD - _E))
    g = _gather_rows(xp, idxf)[:_E]

    w1 = layers[0]['W']
    c1 = w1.shape[1]
    w1p = jnp.zeros((2 * dpsc, c1), jnp.float32)
    w1p = w1p.at[:d].set(w1[:d]).at[dpsc:dpsc + d].set(w1[d:])
    b1 = layers[0]['b'][None]
    xb = xp[:n]
    grid = (_E // _EB,)
    gspec = pl.BlockSpec((_EB, dpsc), lambda i: (i, 0))
    xspec = pl.BlockSpec((_PB, dpsc), lambda i: (i, 0))
    stat = pl.BlockSpec((1, c1), lambda i: (0, 0))

    s1, ss1 = pl.pallas_call(
        _p1_body, grid=grid,
        in_specs=[gspec, xspec, _full2(w1p), _full2(b1)],
        out_specs=[stat, stat],
        out_shape=[jax.ShapeDtypeStruct((1, c1), jnp.float32)] * 2,
    )(g, xb, w1p, b1)
    m1 = s1 / _E
    v1 = ss1 / _E - m1 * m1
    bn1 = jnp.concatenate(
        [m1, v1, layers[0]['gamma'][None], layers[0]['beta'][None]], 0)

    w2 = layers[1]['W']
    c2 = w2.shape[1]
    b2 = layers[1]['b'][None]
    stat2 = pl.BlockSpec((1, c2), lambda i: (0, 0))
    h2, s2, ss2 = pl.pallas_call(
        _p2_body, grid=grid,
        in_specs=[gspec, xspec, _full2(w1p), _full2(b1), _full2(bn1),
                  _full2(w2), _full2(b2)],
        out_specs=[pl.BlockSpec((_EB, c2), lambda i: (i, 0)), stat2, stat2],
        out_shape=[jax.ShapeDtypeStruct((_E, c2), jnp.float32),
                   jax.ShapeDtypeStruct((1, c2), jnp.float32),
                   jax.ShapeDtypeStruct((1, c2), jnp.float32)],
    )(g, xb, w1p, b1, bn1, w2, b2)
    m2 = s2 / _E
    v2 = ss2 / _E - m2 * m2
    bn2 = jnp.concatenate(
        [m2, v2, layers[1]['gamma'][None], layers[1]['beta'][None]], 0)

    h2spec = pl.BlockSpec((_EB, c2), lambda i: (i, 0))
    ospec = pl.BlockSpec((_PB, c2), lambda i: (i, 0))
    if epi is None:
        return pl.pallas_call(
            _p3_pool_body, grid=grid,
            in_specs=[h2spec, _full2(bn2)],
            out_specs=ospec,
            out_shape=jax.ShapeDtypeStruct((n, c2), jnp.float32),
        )(h2, bn2)

    lt, l0, l1 = epi
    ltwp = jnp.zeros((dpsc, c2), jnp.float32).at[:d].set(lt['W'])
    return pl.pallas_call(
        _p3_epi_body, grid=grid,
        in_specs=[h2spec, _full2(bn2), xspec, _full2(ltwp),
                  _full2(lt['b'][None]), _full2(l0['W']),
                  _full2(l0['b'][None]), _full2(l1['W']),
                  _full2(l1['b'][None])],
        out_specs=ospec,
        out_shape=jax.ShapeDtypeStruct((n, c2), jnp.float32),
    )(h2, bn2, xb, ltwp, lt['b'][None], l0['W'], l0['b'][None],
      l1['W'], l1['b'][None])


def _t2_body(x_ref, w_ref, b_ref, s_ref, ss_ref, mx_ref, mn_ref):
    y = jnp.dot(x_ref[...], w_ref[...]) + b_ref[...]

    @pl.when(pl.program_id(0) == 0)
    def _():
        s_ref[...] = jnp.zeros_like(s_ref)
        ss_ref[...] = jnp.zeros_like(ss_ref)
        mx_ref[...] = jnp.full_like(mx_ref, -jnp.inf)
        mn_ref[...] = jnp.full_like(mn_ref, jnp.inf)

    s_ref[...] += jnp.sum(y, axis=0, keepdims=True)
    ss_ref[...] += jnp.sum(y * y, axis=0, keepdims=True)
    mx_ref[...] = jnp.maximum(mx_ref[...], jnp.max(y, axis=0, keepdims=True))
    mn_ref[...] = jnp.minimum(mn_ref[...], jnp.min(y, axis=0, keepdims=True))


def _t2_pool(x1, p):
    n, din = x1.shape
    dout = p['W'].shape[1]
    rb = 1000
    stat = pl.BlockSpec((1, dout), lambda i: (0, 0))
    s, ss, mx, mn = pl.pallas_call(
        _t2_body, grid=(n // rb,),
        in_specs=[pl.BlockSpec((rb, din), lambda i: (i, 0)),
                  _full2(p['W']), _full2(p['b'][None])],
        out_specs=[stat] * 4,
        out_shape=[jax.ShapeDtypeStruct((1, dout), jnp.float32)] * 4,
    )(x1, p['W'], p['b'][None])
    m = s / n
    v = ss / n - m * m
    g = p['gamma'][None]
    pooled = jnp.where(g >= 0, mx, mn)
    return _leaky(g * (pooled - m) / jnp.sqrt(v + 1e-5) + p['beta'][None])


def _head_body(p_ref, w1_ref, b1_ref, w2_ref, b2_ref, w4_ref, b4_ref, o_ref):
    h = _leaky(jnp.dot(p_ref[...], w1_ref[...]) + b1_ref[...])
    h = _leaky(jnp.dot(h, w2_ref[...]) + b2_ref[...])
    o_ref[...] = jnp.dot(h, w4_ref[...]) + b4_ref[...]


def _head(pooled, t3, t4):
    args = (pooled, t3[0]['W'], t3[0]['b'][None], t3[1]['W'], t3[1]['b'][None],
            t4['W'], t4['b'][None])
    return pl.pallas_call(
        _head_body,
        in_specs=[pl.BlockSpec(a.shape, lambda: (0, 0)) for a in args],
        out_specs=pl.BlockSpec((1, 9), lambda: (0, 0)),
        out_shape=jax.ShapeDtypeStruct((1, 9), jnp.float32),
    )(*args)


def kernel(positions, features, batch_indices, params):
    del batch_indices
    x1 = _conv_block(positions, params['t1'], None)
    pooled = _t2_pool(x1, params['t2'][0])
    x9 = _head(pooled, params['t3'], params['t4'])
    t = x9.reshape(3, 3)
    x0 = (positions[:, 0:1] * t[0][None, :]
          + positions[:, 1:2] * t[1][None, :]
          + positions[:, 2:3] * t[2][None, :])
    x = jnp.concatenate([x0, features], axis=-1)
    for i in range(2):
        x = _conv_block(x, params['convs'][i],
                        (params['lin_transform'][i],) +
                        tuple(params['lin_layers'][i]))
    return x
